# Initial kernel scaffold; baseline (speedup 1.0000x reference)
import jax
import jax.numpy as jnp
from jax.experimental import pallas as pl


def kernel(x, edge_index, edge_attr, batch, W_emb, b_emb, W1, b1, W2, W3, b3, Wl1, bl1, Wl2, bl2):
    raise NotImplementedError("write your pallas kernel here")


# speedup vs baseline: 4.4732x; 4.4732x over previous
import functools

import jax
import jax.numpy as jnp
from jax import lax
from jax.experimental import pallas as pl
from jax.experimental.pallas import tpu as pltpu
from jax.experimental.pallas import tpu_sc as plsc

N = 50000
E = 800000
G = 512
H = 64
NHALF = N // 2
CS = 80
NCHUNK = E // CS
KB = 40
NBLKTOT = NCHUNK // KB
ZR = 200

_mesh = plsc.VectorSubcoreMesh(core_axis_name="c", subcore_axis_name="s")
_sc_params = pltpu.CompilerParams(use_tc_tiling_on_sc=False)


def _zero_rows(buf, nrows):
    def body(k, _):
        r = buf.at[k]
        for g in range(H // 16):
            r[pl.ds(g * 16, 16)] = jnp.zeros((16,), jnp.float32)
        return 0
    lax.fori_loop(0, nrows, body, 0)


@functools.partial(
    pl.kernel,
    out_type=jax.ShapeDtypeStruct((N, H), jnp.float32),
    mesh=_mesh,
    compiler_params=_sc_params,
    scratch_types=[
        pltpu.VMEM((KB, CS), jnp.int32),
        pltpu.VMEM((KB, CS), jnp.int32),
        pltpu.VMEM((KB, CS), jnp.float32),
        pltpu.VMEM((CS,), jnp.int32),
        pltpu.VMEM((CS, H), jnp.float32),
        pltpu.VMEM((ZR, H), jnp.float32),
        pltpu.VMEM_SHARED((NHALF, H), jnp.float32),
        pltpu.SemaphoreType.DMA,
    ],
)
def _spmm(h_hbm, src_hbm, dst_hbm, ea_hbm, out_hbm,
          idxb, dstb, eab, loc_v, rows_v, zbuf, acc_sh, sem):
    c = lax.axis_index("c")
    s = lax.axis_index("s")
    base = c * NHALF

    _zero_rows(zbuf, ZR)

    def zchunk(i, _):
        j = s + 16 * i
        @pl.when(j < NHALF // ZR)
        def _():
            pltpu.sync_copy(zbuf, acc_sh.at[pl.ds(j * ZR, ZR)])
        return 0
    lax.fori_loop(0, 8, zchunk, 0)
    plsc.subcore_barrier()

    def block(i, _):
        b = s + 16 * i
        @pl.when(b < NBLKTOT)
        def _():
            rb = b * KB
            pltpu.sync_copy(src_hbm.at[pl.ds(rb, KB)], idxb)
            pltpu.sync_copy(dst_hbm.at[pl.ds(rb, KB)], dstb)
            pltpu.sync_copy(ea_hbm.at[pl.ds(rb, KB)], eab)

            def chunk(j, _):
                pltpu.async_copy(h_hbm.at[idxb.at[j]], rows_v, sem).wait()
                for g in range(CS // 16):
                    d = dstb[j, pl.ds(g * 16, 16)]
                    e = eab[j, pl.ds(g * 16, 16)]
                    inr = (d >= base) & (d < base + NHALF)
                    loc_v[pl.ds(g * 16, 16)] = jnp.where(inr, d - base, 0)
                    ev = jnp.where(inr, e, jnp.float32(0.0))
                    for t in range(16):
                        sv = ev[t]
                        r = rows_v.at[g * 16 + t]
                        for q in range(H // 16):
                            r[pl.ds(q * 16, 16)] = r[pl.ds(q * 16, 16)] * sv
                pltpu.sync_copy(rows_v, acc_sh.at[loc_v], add=True)
                return 0
            lax.fori_loop(0, KB, chunk, 0)
        return 0
    lax.fori_loop(0, 16, block, 0)
    plsc.subcore_barrier()

    def wchunk(i, _):
        j = s + 16 * i
        @pl.when(j < NHALF // ZR)
        def _():
            pltpu.sync_copy(acc_sh.at[pl.ds(j * ZR, ZR)], zbuf)
            pltpu.sync_copy(zbuf, out_hbm.at[pl.ds(base + j * ZR, ZR)])
        return 0
    lax.fori_loop(0, 8, wchunk, 0)


@functools.partial(
    pl.kernel,
    out_type=jax.ShapeDtypeStruct((N,), jnp.float32),
    mesh=_mesh,
    compiler_params=_sc_params,
    scratch_types=[
        pltpu.VMEM((KB, CS), jnp.int32),
        pltpu.VMEM((KB, CS), jnp.float32),
        pltpu.VMEM((CS,), jnp.int32),
        pltpu.VMEM((CS,), jnp.float32),
        pltpu.VMEM((ZR + 8,), jnp.float32),
        pltpu.VMEM_SHARED((NHALF,), jnp.float32),
    ],
)
def _deg(dst_hbm, ea_hbm, out_hbm, dstb, eab, loc_v, es_v, zvec, deg_sh):
    c = lax.axis_index("c")
    s = lax.axis_index("s")
    base = c * NHALF

    for g in range((ZR + 8) // 16):
        zvec[pl.ds(g * 16, 16)] = jnp.zeros((16,), jnp.float32)

    def zchunk(i, _):
        j = s + 16 * i
        @pl.when(j < NHALF // ZR)
        def _():
            pltpu.sync_copy(zvec.at[pl.ds(0, ZR)],
                            deg_sh.at[pl.ds(j * ZR, ZR)])
        return 0
    lax.fori_loop(0, 8, zchunk, 0)
    plsc.subcore_barrier()

    def block(i, _):
        b = s + 16 * i
        @pl.when(b < NBLKTOT)
        def _():
            rb = b * KB
            pltpu.sync_copy(dst_hbm.at[pl.ds(rb, KB)], dstb)
            pltpu.sync_copy(ea_hbm.at[pl.ds(rb, KB)], eab)

            def chunk(j, _):
                for g in range(CS // 16):
                    d = dstb[j, pl.ds(g * 16, 16)]
                    e = eab[j, pl.ds(g * 16, 16)]
                    inr = (d >= base) & (d < base + NHALF)
                    loc_v[pl.ds(g * 16, 16)] = jnp.where(inr, d - base, 0)
                    es_v[pl.ds(g * 16, 16)] = jnp.where(inr, e,
                                                        jnp.float32(0.0))
                pltpu.sync_copy(es_v, deg_sh.at[loc_v], add=True)
                return 0
            lax.fori_loop(0, KB, chunk, 0)
        return 0
    lax.fori_loop(0, 16, block, 0)
    plsc.subcore_barrier()

    def wchunk(i, _):
        j = s + 16 * i
        @pl.when(j < NHALF // ZR)
        def _():
            pltpu.sync_copy(deg_sh.at[pl.ds(j * ZR, ZR)],
                            zvec.at[pl.ds(0, ZR)])
            pltpu.sync_copy(zvec.at[pl.ds(0, ZR)],
                            out_hbm.at[pl.ds(c * NHALF + j * ZR, ZR)])
        return 0
    lax.fori_loop(0, 8, wchunk, 0)


R = 1000


def _embed_body(x_ref, w_ref, b_ref, o_ref):
    o_ref[...] = jnp.dot(x_ref[...], w_ref[...],
                         preferred_element_type=jnp.float32) + b_ref[...]


def _embed(x, W_emb, b_emb):
    return pl.pallas_call(
        _embed_body,
        grid=(N // R,),
        in_specs=[
            pl.BlockSpec((R, 4), lambda i: (i, 0)),
            pl.BlockSpec((4, H), lambda i: (0, 0)),
            pl.BlockSpec((1, H), lambda i: (0, 0)),
        ],
        out_specs=pl.BlockSpec((R, H), lambda i: (i, 0)),
        out_shape=jax.ShapeDtypeStruct((N, H), jnp.float32),
    )(x, W_emb, b_emb)


def _update_body(a_ref, h_ref, d_ref, w1_ref, b1_ref, w2_ref, w3_ref, b3_ref,
                 o_ref):
    h = h_ref[...]
    agg = (jnp.dot(a_ref[...], w1_ref[...], preferred_element_type=jnp.float32)
           + d_ref[...] * (b1_ref[...]
                           - jnp.dot(h, w2_ref[...],
                                     preferred_element_type=jnp.float32)))
    o_ref[...] = jnp.maximum(
        agg + jnp.dot(h, w3_ref[...], preferred_element_type=jnp.float32)
        + b3_ref[...], 0.0)


def _update(A, h, degw, W1i, b1i, W2i, W3i, b3i):
    return pl.pallas_call(
        _update_body,
        grid=(N // R,),
        in_specs=[
            pl.BlockSpec((R, H), lambda i: (i, 0)),
            pl.BlockSpec((R, H), lambda i: (i, 0)),
            pl.BlockSpec((R, 1), lambda i: (i, 0)),
            pl.BlockSpec((H, H), lambda i: (0, 0)),
            pl.BlockSpec((1, H), lambda i: (0, 0)),
            pl.BlockSpec((H, H), lambda i: (0, 0)),
            pl.BlockSpec((H, H), lambda i: (0, 0)),
            pl.BlockSpec((1, H), lambda i: (0, 0)),
        ],
        out_specs=pl.BlockSpec((R, H), lambda i: (i, 0)),
        out_shape=jax.ShapeDtypeStruct((N, H), jnp.float32),
    )(A, h, degw, W1i, b1i, W2i, W3i, b3i)


def _pool_body(h_ref, bat_ref, ps_ref, pc_ref):
    i = pl.program_id(0)
    onehot = (bat_ref[...] == lax.broadcasted_iota(jnp.int32, (R, G), 1)
              ).astype(jnp.float32)
    ps = lax.dot_general(onehot, h_ref[...], (((0,), (0,)), ((), ())),
                         preferred_element_type=jnp.float32)
    pc = lax.dot_general(onehot, jnp.ones((R, 1), jnp.float32),
                         (((0,), (0,)), ((), ())),
                         preferred_element_type=jnp.float32)

    @pl.when(i == 0)
    def _():
        ps_ref[...] = ps
        pc_ref[...] = pc

    @pl.when(i > 0)
    def _():
        ps_ref[...] += ps
        pc_ref[...] += pc


def _pool(h, bat):
    return pl.pallas_call(
        _pool_body,
        grid=(N // R,),
        in_specs=[
            pl.BlockSpec((R, H), lambda i: (i, 0)),
            pl.BlockSpec((R, 1), lambda i: (i, 0)),
        ],
        out_specs=[
            pl.BlockSpec((G, H), lambda i: (0, 0)),
            pl.BlockSpec((G, 1), lambda i: (0, 0)),
        ],
        out_shape=[jax.ShapeDtypeStruct((G, H), jnp.float32),
                   jax.ShapeDtypeStruct((G, 1), jnp.float32)],
    )(h, bat)


def _head_body(ps_ref, pc_ref, w1_ref, b1_ref, w2_ref, b2_ref, o_ref):
    gx = ps_ref[...] / jnp.maximum(pc_ref[...], 1.0)
    t = jnp.maximum(
        jnp.dot(gx, w1_ref[...], preferred_element_type=jnp.float32)
        + b1_ref[...], 0.0)
    o_ref[...] = jnp.dot(t, w2_ref[...],
                         preferred_element_type=jnp.float32) + b2_ref[...]


def _head(psum, pcnt, Wl1, bl1, Wl2, bl2):
    return pl.pallas_call(
        _head_body,
        out_shape=jax.ShapeDtypeStruct((G, 3), jnp.float32),
    )(psum, pcnt, Wl1, bl1, Wl2, bl2)


def kernel(x, edge_index, edge_attr, batch,
           W_emb, b_emb, W1, b1, W2, W3, b3, Wl1, bl1, Wl2, bl2):
    src2 = edge_index[0].reshape(NCHUNK, CS)
    dst2 = edge_index[1].reshape(NCHUNK, CS)
    ea2 = edge_attr.reshape(NCHUNK, CS)

    h = _embed(x, W_emb, b_emb.reshape(1, H))
    degw = _deg(dst2, ea2).reshape(N, 1)
    for i in range(3):
        A = _spmm(h, src2, dst2, ea2)
        h = _update(A, h, degw, W1[i], b1[i].reshape(1, H),
                    W2[i], W3[i], b3[i].reshape(1, H))
    psum, pcnt = _pool(h, batch.reshape(N, 1))
    return _head(psum, pcnt, Wl1, bl1.reshape(1, H),
                 Wl2, bl2.reshape(1, 3))
